# baseline (speedup 1.0000x reference)
import functools

import jax
import jax.numpy as jnp
from jax import lax
from jax.experimental import pallas as pl
from jax.experimental.pallas import tpu as pltpu
from jax.experimental.pallas import tpu_sc as plsc

A_ROWS = 100000
EMB = 16
N_EDGES = 3200000
NC, NS = 2, 16
NW = NC * NS

CH_EDGES = 1024
NTILES = CH_EDGES // 128
NCHUNKS = N_EDGES // CH_EDGES
K_ITERS = -(-NCHUNKS // NW)
HALF = N_EDGES * 8


def _body(x_hbm, w_hbm, out_hbm, idx_v, rows_v, outt_v, sem_i, sem_g, sem_o):
    wid = lax.axis_index("s") * NC + lax.axis_index("c")
    iota = lax.iota(jnp.int32, 16)
    voff = (iota >> 3) * (NTILES * 1024) + (iota & 7) * 128
    vofft = [voff + t * 1024 for t in range(NTILES)]

    def chunk_of(k):
        return wid + NW * k

    def idx_start(k, b):
        pltpu.async_copy(x_hbm.at[pl.ds(chunk_of(k) * 2048, 2048)],
                         idx_v.at[b], sem_i[b])

    def idx_wait(b):
        pltpu.make_async_copy(x_hbm.at[pl.ds(0, 2048)],
                              idx_v.at[b], sem_i[b]).wait()

    def offset_pass(b):
        def off_body(i, cc):
            off = (i >> 3) * 256 + 128 + ((i & 7) << 4)
            idx_v[b, pl.ds(off, 16)] = idx_v[b, pl.ds(off, 16)] + A_ROWS
            return cc
        lax.fori_loop(0, 64, off_body, 0, unroll=8)

    def gather_start(b):
        pltpu.async_copy(w_hbm.at[idx_v.at[b, pl.ds(0, 1024)]],
                         rows_v.at[b, pl.ds(0, 1024)], sem_g[b])
        pltpu.async_copy(w_hbm.at[idx_v.at[b, pl.ds(1024, 1024)]],
                         rows_v.at[b, pl.ds(1024, 1024)], sem_g[b])

    def gather_wait(b):
        for _ in range(2):
            pltpu.make_async_copy(w_hbm.at[idx_v.at[b, pl.ds(0, 1024)]],
                                  rows_v.at[b, pl.ds(0, 1024)],
                                  sem_g[b]).wait()

    def add_scatter(b):
        for t in range(NTILES):
            base = t * 256

            def inner(jl, posv):
                val = rows_v[b, base + jl] + rows_v[b, base + 128 + jl]
                plsc.store_scatter(outt_v.at[b], [posv], val)
                return posv + 1

            lax.fori_loop(0, 128, inner, vofft[t], unroll=8)

    def out_start(k, b):
        c = chunk_of(k)
        pltpu.async_copy(outt_v.at[b, pl.ds(0, 8192)],
                         out_hbm.at[pl.ds(c * 8192, 8192)], sem_o[b])
        pltpu.async_copy(outt_v.at[b, pl.ds(8192, 8192)],
                         out_hbm.at[pl.ds(HALF + c * 8192, 8192)], sem_o[b])

    def out_wait(b):
        for _ in range(2):
            pltpu.make_async_copy(outt_v.at[b, pl.ds(0, 8192)],
                                  out_hbm.at[pl.ds(0, 8192)],
                                  sem_o[b]).wait()

    @pl.when(chunk_of(0) < NCHUNKS)
    def _():
        idx_start(0, 0)

    def block(k, b, carry):
        valid_k = chunk_of(k) < NCHUNKS

        @pl.when(valid_k)
        def _():
            idx_wait(b)
            offset_pass(b)
            gather_start(b)

        @pl.when((k >= 1) & (chunk_of(k - 1) < NCHUNKS))
        def _():
            gather_wait(1 - b)
            @pl.when(k >= 3)
            def _():
                out_wait(1 - b)
            add_scatter(1 - b)
            out_start(k - 1, 1 - b)

        @pl.when(chunk_of(k + 1) < NCHUNKS)
        def _():
            idx_start(k + 1, 1 - b)

        return carry

    def block_pair(m, carry):
        k = 2 * m
        block(k, 0, carry)
        block(k + 1, 1, carry)
        return carry

    lax.fori_loop(0, K_ITERS // 2 + 1, block_pair, 0)

    @pl.when(chunk_of(K_ITERS - 2) < NCHUNKS)
    def _():
        out_wait((K_ITERS - 2) % 2)

    @pl.when(chunk_of(K_ITERS - 1) < NCHUNKS)
    def _():
        out_wait((K_ITERS - 1) % 2)


_gather_sum = functools.partial(
    pl.kernel,
    mesh=plsc.VectorSubcoreMesh(core_axis_name="c", subcore_axis_name="s"),
    out_type=jax.ShapeDtypeStruct((N_EDGES * EMB,), jnp.float32),
    scratch_types=[
        pltpu.VMEM((2, 2 * CH_EDGES), jnp.int32),
        pltpu.VMEM((2, 2 * CH_EDGES, EMB), jnp.float32),
        pltpu.VMEM((2, CH_EDGES * EMB), jnp.float32),
        [pltpu.SemaphoreType.DMA] * 2,
        [pltpu.SemaphoreType.DMA] * 2,
        [pltpu.SemaphoreType.DMA] * 2,
    ],
    compiler_params=pltpu.CompilerParams(use_tc_tiling_on_sc=False,
                                         needs_layout_passes=False),
)(_body)


def kernel(edge_attr, W0, W1):
    if edge_attr.ndim == 1:
        edge_attr = edge_attr[:, None]
    x1d = (edge_attr.astype(jnp.int32)
           .reshape(N_EDGES // 128, 128, 2)
           .transpose(0, 2, 1)
           .reshape(-1))
    w = jnp.concatenate([W0, W1], axis=0)
    outf = _gather_sum(x1d, w)
    return (outf.reshape(2, N_EDGES // 128, 8, 128)
            .transpose(1, 3, 0, 2)
            .reshape(N_EDGES, EMB))


# speedup vs baseline: 1.0017x; 1.0017x over previous
import functools

import jax
import jax.numpy as jnp
from jax import lax
from jax.experimental import pallas as pl
from jax.experimental.pallas import tpu as pltpu
from jax.experimental.pallas import tpu_sc as plsc

A_ROWS = 100000
EMB = 16
N_EDGES = 3200000
NC, NS = 2, 16
NW = NC * NS

CH_EDGES = 1024
NTILES = CH_EDGES // 128
NCHUNKS = N_EDGES // CH_EDGES
K_ITERS = -(-NCHUNKS // NW)
HALF = N_EDGES * 8


def _body(x_hbm, w_hbm, out_hbm, idx_v, rows_v, outt_v, sem_i, sem_g, sem_o):
    wid = lax.axis_index("s") * NC + lax.axis_index("c")
    iota = lax.iota(jnp.int32, 16)
    voff = (iota >> 3) * (NTILES * 1024) + (iota & 7) * 128
    vofft = [voff + t * 1024 for t in range(NTILES)]

    def chunk_of(k):
        return wid + NW * k

    def idx_start(k, b):
        pltpu.async_copy(x_hbm.at[pl.ds(chunk_of(k) * 2048, 2048)],
                         idx_v.at[b], sem_i[b])

    def idx_wait(b):
        pltpu.make_async_copy(x_hbm.at[pl.ds(0, 2048)],
                              idx_v.at[b], sem_i[b]).wait()

    def offset_pass(b):
        def off_body(i, cc):
            off_base = i * 256 + 128
            for u in range(8):
                off = off_base + u * 16
                idx_v[b, pl.ds(off, 16)] = idx_v[b, pl.ds(off, 16)] + A_ROWS
            return cc
        lax.fori_loop(0, 8, off_body, 0)

    def gather_start(b):
        pltpu.async_copy(w_hbm.at[idx_v.at[b]], rows_v.at[b], sem_g[b])

    def gather_wait(b):
        pltpu.make_async_copy(w_hbm.at[idx_v.at[b]],
                              rows_v.at[b], sem_g[b]).wait()

    def add_scatter(b):
        for t in range(NTILES):
            base = t * 256

            def inner(g, posv):
                jl = g * 8
                for u in range(8):
                    val = (rows_v[b, base + jl + u]
                           + rows_v[b, base + 128 + jl + u])
                    plsc.store_scatter(outt_v.at[b], [posv + u], val)
                return posv + 8

            lax.fori_loop(0, 16, inner, vofft[t])

    def out_start(k, b):
        c = chunk_of(k)
        pltpu.async_copy(outt_v.at[b, pl.ds(0, 8192)],
                         out_hbm.at[pl.ds(c * 8192, 8192)], sem_o[b])
        pltpu.async_copy(outt_v.at[b, pl.ds(8192, 8192)],
                         out_hbm.at[pl.ds(HALF + c * 8192, 8192)], sem_o[b])

    def out_wait(b):
        for _ in range(2):
            pltpu.make_async_copy(outt_v.at[b, pl.ds(0, 8192)],
                                  out_hbm.at[pl.ds(0, 8192)],
                                  sem_o[b]).wait()

    @pl.when(chunk_of(0) < NCHUNKS)
    def _():
        idx_start(0, 0)

    def block(k, b, carry):
        valid_k = chunk_of(k) < NCHUNKS

        @pl.when(valid_k)
        def _():
            idx_wait(b)
            offset_pass(b)
            gather_start(b)

        @pl.when((k >= 1) & (chunk_of(k - 1) < NCHUNKS))
        def _():
            gather_wait(1 - b)
            @pl.when(k >= 3)
            def _():
                out_wait(1 - b)
            add_scatter(1 - b)
            out_start(k - 1, 1 - b)

        @pl.when(chunk_of(k + 1) < NCHUNKS)
        def _():
            idx_start(k + 1, 1 - b)

        return carry

    def block_pair(m, carry):
        k = 2 * m
        block(k, 0, carry)
        block(k + 1, 1, carry)
        return carry

    lax.fori_loop(0, K_ITERS // 2 + 1, block_pair, 0)

    @pl.when(chunk_of(K_ITERS - 2) < NCHUNKS)
    def _():
        out_wait((K_ITERS - 2) % 2)

    @pl.when(chunk_of(K_ITERS - 1) < NCHUNKS)
    def _():
        out_wait((K_ITERS - 1) % 2)


_gather_sum = functools.partial(
    pl.kernel,
    mesh=plsc.VectorSubcoreMesh(core_axis_name="c", subcore_axis_name="s"),
    out_type=jax.ShapeDtypeStruct((N_EDGES * EMB,), jnp.float32),
    scratch_types=[
        pltpu.VMEM((2, 2 * CH_EDGES), jnp.int32),
        pltpu.VMEM((2, 2 * CH_EDGES, EMB), jnp.float32),
        pltpu.VMEM((2, CH_EDGES * EMB), jnp.float32),
        [pltpu.SemaphoreType.DMA] * 2,
        [pltpu.SemaphoreType.DMA] * 2,
        [pltpu.SemaphoreType.DMA] * 2,
    ],
    compiler_params=pltpu.CompilerParams(use_tc_tiling_on_sc=False,
                                         needs_layout_passes=False),
)(_body)


def kernel(edge_attr, W0, W1):
    if edge_attr.ndim == 1:
        edge_attr = edge_attr[:, None]
    x1d = (edge_attr.astype(jnp.int32)
           .reshape(N_EDGES // 128, 128, 2)
           .transpose(0, 2, 1)
           .reshape(-1))
    w = jnp.concatenate([W0, W1], axis=0)
    outf = _gather_sum(x1d, w)
    return (outf.reshape(2, N_EDGES // 128, 8, 128)
            .transpose(1, 3, 0, 2)
            .reshape(N_EDGES, EMB))


# speedup vs baseline: 1.2585x; 1.2564x over previous
import functools

import jax
import jax.numpy as jnp
from jax import lax
from jax.experimental import pallas as pl
from jax.experimental.pallas import tpu as pltpu
from jax.experimental.pallas import tpu_sc as plsc

A_ROWS = 100000
EMB = 16
N_EDGES = 3200000
NC, NS = 2, 16
NW = NC * NS

CH_EDGES = 1024
NTILES = CH_EDGES // 128
NCHUNKS = N_EDGES // CH_EDGES
K_ITERS = -(-NCHUNKS // NW)
HALF = N_EDGES * 8

ROW_STRIDE = 129
TILE_STRIDE = 1032
BAND_STRIDE = 8264
PAD_WORDS = 2 * BAND_STRIDE


def _body(x_hbm, w_hbm, out_hbm, idx_v, rows_v, outp_v, oute_v,
          sem_i, sem_g, sem_o):
    wid = lax.axis_index("s") * NC + lax.axis_index("c")
    iota = lax.iota(jnp.int32, 16)
    voff = (iota >> 3) * BAND_STRIDE + (iota & 7) * ROW_STRIDE
    vofft = [voff + t * TILE_STRIDE for t in range(NTILES)]

    def chunk_of(k):
        return wid + NW * k

    def idx_start(k, b):
        pltpu.async_copy(x_hbm.at[pl.ds(chunk_of(k) * 2048, 2048)],
                         idx_v.at[b], sem_i[b])

    def idx_wait(b):
        pltpu.make_async_copy(x_hbm.at[pl.ds(0, 2048)],
                              idx_v.at[b], sem_i[b]).wait()

    def offset_pass(b):
        def off_body(i, cc):
            off_base = i * 256 + 128
            for u in range(8):
                off = off_base + u * 16
                idx_v[b, pl.ds(off, 16)] = idx_v[b, pl.ds(off, 16)] + A_ROWS
            return cc
        lax.fori_loop(0, 8, off_body, 0)

    def gather_start(b):
        pltpu.async_copy(w_hbm.at[idx_v.at[b]], rows_v.at[b], sem_g[b])

    def gather_wait(b):
        pltpu.make_async_copy(w_hbm.at[idx_v.at[b]],
                              rows_v.at[b], sem_g[b]).wait()

    def add_scatter(b):
        for t in range(NTILES):
            base = t * 256

            def inner(g, posv):
                jl = g * 8
                for u in range(8):
                    val = (rows_v[b, base + jl + u]
                           + rows_v[b, base + 128 + jl + u])
                    plsc.store_scatter(outp_v, [posv + u], val)
                return posv + 8

            lax.fori_loop(0, 16, inner, vofft[t])

    def compact():
        def row_body(r, cc):
            src = ((r >> 6) * BAND_STRIDE + ((r >> 3) & 7) * TILE_STRIDE
                   + (r & 7) * ROW_STRIDE)
            dst = r * 128
            for u in range(8):
                oute_v[pl.ds(dst + u * 16, 16)] = outp_v[pl.ds(src + u * 16, 16)]
            return cc
        lax.fori_loop(0, 128, row_body, 0)

    def out_start(k):
        c = chunk_of(k)
        pltpu.async_copy(oute_v.at[pl.ds(0, 8192)],
                         out_hbm.at[pl.ds(c * 8192, 8192)], sem_o)
        pltpu.async_copy(oute_v.at[pl.ds(8192, 8192)],
                         out_hbm.at[pl.ds(HALF + c * 8192, 8192)], sem_o)

    def out_wait():
        for _ in range(2):
            pltpu.make_async_copy(oute_v.at[pl.ds(0, 8192)],
                                  out_hbm.at[pl.ds(0, 8192)], sem_o).wait()

    @pl.when(chunk_of(0) < NCHUNKS)
    def _():
        idx_start(0, 0)

    def block(k, b, carry):
        @pl.when(chunk_of(k) < NCHUNKS)
        def _():
            idx_wait(b)
            offset_pass(b)
            gather_start(b)

        @pl.when((k >= 1) & (chunk_of(k - 1) < NCHUNKS))
        def _():
            gather_wait(1 - b)
            add_scatter(1 - b)
            @pl.when(k >= 2)
            def _():
                out_wait()
            compact()
            out_start(k - 1)

        @pl.when(chunk_of(k + 1) < NCHUNKS)
        def _():
            idx_start(k + 1, 1 - b)

        return carry

    def block_pair(m, carry):
        k = 2 * m
        block(k, 0, carry)
        block(k + 1, 1, carry)
        return carry

    lax.fori_loop(0, K_ITERS // 2 + 1, block_pair, 0)

    out_wait()


_gather_sum = functools.partial(
    pl.kernel,
    mesh=plsc.VectorSubcoreMesh(core_axis_name="c", subcore_axis_name="s"),
    out_type=jax.ShapeDtypeStruct((N_EDGES * EMB,), jnp.float32),
    scratch_types=[
        pltpu.VMEM((2, 2 * CH_EDGES), jnp.int32),
        pltpu.VMEM((2, 2 * CH_EDGES, EMB), jnp.float32),
        pltpu.VMEM((PAD_WORDS,), jnp.float32),
        pltpu.VMEM((CH_EDGES * EMB,), jnp.float32),
        [pltpu.SemaphoreType.DMA] * 2,
        [pltpu.SemaphoreType.DMA] * 2,
        pltpu.SemaphoreType.DMA,
    ],
    compiler_params=pltpu.CompilerParams(use_tc_tiling_on_sc=False,
                                         needs_layout_passes=False),
)(_body)


def kernel(edge_attr, W0, W1):
    if edge_attr.ndim == 1:
        edge_attr = edge_attr[:, None]
    x1d = (edge_attr.astype(jnp.int32)
           .reshape(N_EDGES // 128, 128, 2)
           .transpose(0, 2, 1)
           .reshape(-1))
    w = jnp.concatenate([W0, W1], axis=0)
    outf = _gather_sum(x1d, w)
    return (outf.reshape(2, N_EDGES // 128, 8, 128)
            .transpose(1, 3, 0, 2)
            .reshape(N_EDGES, EMB))


# speedup vs baseline: 1.6898x; 1.3427x over previous
import functools

import jax
import jax.numpy as jnp
from jax import lax
from jax.experimental import pallas as pl
from jax.experimental.pallas import tpu as pltpu
from jax.experimental.pallas import tpu_sc as plsc

A_ROWS = 100000
EMB = 16
N_EDGES = 3200000
NC, NS = 2, 16
NW = NC * NS

CH_EDGES = 1024
NTILES = CH_EDGES // 128
NCHUNKS = N_EDGES // CH_EDGES
K_ITERS = -(-NCHUNKS // NW)
OUT_ROWS = N_EDGES * EMB // 128
BAND_ROWS = OUT_ROWS // 2


def _body(x_hbm, w_hbm, out_hbm, idx_v, rows_v, outp_v, sem_i, sem_g, sem_o):
    wid = lax.axis_index("s") * NC + lax.axis_index("c")
    iota = lax.iota(jnp.int32, 16)
    zero16 = iota * 0
    vrow = (iota >> 3) * 64 + (iota & 7)
    vrowt = [vrow + t * 8 for t in range(NTILES)]

    def chunk_of(k):
        return wid + NW * k

    def idx_start(k, b):
        pltpu.async_copy(x_hbm.at[pl.ds(chunk_of(k) * 2048, 2048)],
                         idx_v.at[b], sem_i[b])

    def idx_wait(b):
        pltpu.make_async_copy(x_hbm.at[pl.ds(0, 2048)],
                              idx_v.at[b], sem_i[b]).wait()

    def offset_pass(b):
        def off_body(i, cc):
            off_base = i * 256 + 128
            for u in range(8):
                off = off_base + u * 16
                idx_v[b, pl.ds(off, 16)] = idx_v[b, pl.ds(off, 16)] + A_ROWS
            return cc
        lax.fori_loop(0, 8, off_body, 0)

    def gather_start(b):
        pltpu.async_copy(w_hbm.at[idx_v.at[b]], rows_v.at[b], sem_g[b])

    def gather_wait(b):
        pltpu.make_async_copy(w_hbm.at[idx_v.at[b]],
                              rows_v.at[b], sem_g[b]).wait()

    def add_scatter(b):
        for t in range(NTILES):
            base = t * 256

            def inner(g, jlv):
                jl = g * 8
                for u in range(8):
                    val = (rows_v[b, base + jl + u]
                           + rows_v[b, base + 128 + jl + u])
                    plsc.store_scatter(outp_v.at[b], [vrowt[t], jlv + u], val)
                return jlv + 8

            lax.fori_loop(0, 16, inner, zero16)

    def out_start(k, b):
        c = chunk_of(k)
        for h in (0, 1):
            pltpu.async_copy(
                outp_v.at[b, pl.ds(h * 64, 64), pl.ds(0, 128)],
                out_hbm.at[pl.ds(h * BAND_ROWS + c * 64, 64)], sem_o[b])

    def out_wait(b):
        for h in (0, 1):
            pltpu.make_async_copy(
                outp_v.at[b, pl.ds(h * 64, 64), pl.ds(0, 128)],
                out_hbm.at[pl.ds(h * 64, 64)], sem_o[b]).wait()

    @pl.when(chunk_of(0) < NCHUNKS)
    def _():
        idx_start(0, 0)

    def block(k, b, carry):
        @pl.when(chunk_of(k) < NCHUNKS)
        def _():
            idx_wait(b)
            offset_pass(b)
            gather_start(b)

        @pl.when((k >= 1) & (chunk_of(k - 1) < NCHUNKS))
        def _():
            gather_wait(1 - b)
            @pl.when(k >= 3)
            def _():
                out_wait(1 - b)
            add_scatter(1 - b)
            out_start(k - 1, 1 - b)

        @pl.when(chunk_of(k + 1) < NCHUNKS)
        def _():
            idx_start(k + 1, 1 - b)

        return carry

    def block_pair(m, carry):
        k = 2 * m
        block(k, 0, carry)
        block(k + 1, 1, carry)
        return carry

    lax.fori_loop(0, K_ITERS // 2 + 1, block_pair, 0)

    out_wait(0)
    out_wait(1)


_gather_sum = functools.partial(
    pl.kernel,
    mesh=plsc.VectorSubcoreMesh(core_axis_name="c", subcore_axis_name="s"),
    out_type=jax.ShapeDtypeStruct((OUT_ROWS, 128), jnp.float32),
    scratch_types=[
        pltpu.VMEM((2, 2 * CH_EDGES), jnp.int32),
        pltpu.VMEM((2, 2 * CH_EDGES, EMB), jnp.float32),
        pltpu.VMEM((2, 128, 129), jnp.float32),
        [pltpu.SemaphoreType.DMA] * 2,
        [pltpu.SemaphoreType.DMA] * 2,
        [pltpu.SemaphoreType.DMA] * 2,
    ],
    compiler_params=pltpu.CompilerParams(use_tc_tiling_on_sc=False,
                                         needs_layout_passes=False),
)(_body)


def kernel(edge_attr, W0, W1):
    if edge_attr.ndim == 1:
        edge_attr = edge_attr[:, None]
    x1d = (edge_attr.astype(jnp.int32)
           .reshape(N_EDGES // 128, 128, 2)
           .transpose(0, 2, 1)
           .reshape(-1))
    w = jnp.concatenate([W0, W1], axis=0)
    out2 = _gather_sum(x1d, w)
    return (out2.reshape(2, N_EDGES // 128, 8, 128)
            .transpose(1, 3, 0, 2)
            .reshape(N_EDGES, EMB))


# speedup vs baseline: 1.6981x; 1.0049x over previous
import functools

import jax
import jax.numpy as jnp
from jax import lax
from jax.experimental import pallas as pl
from jax.experimental.pallas import tpu as pltpu
from jax.experimental.pallas import tpu_sc as plsc

A_ROWS = 100000
EMB = 16
N_EDGES = 3200000
NC, NS = 2, 16
NW = NC * NS

CH_EDGES = 1024
NTILES = CH_EDGES // 128
NCHUNKS = N_EDGES // CH_EDGES
K_ITERS = -(-NCHUNKS // NW)
OUT_ROWS = N_EDGES * EMB // 128
BAND_ROWS = OUT_ROWS // 2


def _body(x_hbm, w_hbm, out_hbm, idx_v, rows_v, outp_v, sem_i, sem_g, sem_o):
    wid = lax.axis_index("s") * NC + lax.axis_index("c")
    iota = lax.iota(jnp.int32, 16)
    zero16 = iota * 0
    vrow = (iota >> 3) * 64 + (iota & 7)
    vrowt = [vrow + t * 8 for t in range(NTILES)]

    c_start = wid * NCHUNKS // NW
    c_end = (wid + 1) * NCHUNKS // NW

    def chunk_of(k):
        return c_start + k

    def idx_start(k, b):
        pltpu.async_copy(x_hbm.at[pl.ds(chunk_of(k) * 2048, 2048)],
                         idx_v.at[b], sem_i[b])

    def idx_wait(b):
        pltpu.make_async_copy(x_hbm.at[pl.ds(0, 2048)],
                              idx_v.at[b], sem_i[b]).wait()

    def offset_pass(b):
        def off_body(i, cc):
            off_base = i * 256 + 128
            for u in range(8):
                off = off_base + u * 16
                idx_v[b, pl.ds(off, 16)] = idx_v[b, pl.ds(off, 16)] + A_ROWS
            return cc
        lax.fori_loop(0, 8, off_body, 0)

    def gather_start(b):
        pltpu.async_copy(w_hbm.at[idx_v.at[b]], rows_v.at[b], sem_g[b])

    def gather_wait(b):
        pltpu.make_async_copy(w_hbm.at[idx_v.at[b]],
                              rows_v.at[b], sem_g[b]).wait()

    def add_scatter(b):
        for t in range(NTILES):
            base = t * 256

            def inner(g, jlv):
                jl = g * 8
                for u in range(8):
                    val = (rows_v[b, base + jl + u]
                           + rows_v[b, base + 128 + jl + u])
                    plsc.store_scatter(outp_v.at[b], [vrowt[t], jlv + u], val)
                return jlv + 8

            lax.fori_loop(0, 16, inner, zero16)

    def out_start(k, b):
        c = chunk_of(k)
        for h in (0, 1):
            pltpu.async_copy(
                outp_v.at[b, pl.ds(h * 64, 64), pl.ds(0, 128)],
                out_hbm.at[pl.ds(h * BAND_ROWS + c * 64, 64)], sem_o[b])

    def out_wait(b):
        for h in (0, 1):
            pltpu.make_async_copy(
                outp_v.at[b, pl.ds(h * 64, 64), pl.ds(0, 128)],
                out_hbm.at[pl.ds(h * 64, 64)], sem_o[b]).wait()

    @pl.when(chunk_of(0) < c_end)
    def _():
        idx_start(0, 0)

    def block(k, b, carry):
        @pl.when(chunk_of(k) < c_end)
        def _():
            idx_wait(b)
            offset_pass(b)
            gather_start(b)

        @pl.when((k >= 1) & (chunk_of(k - 1) < c_end))
        def _():
            gather_wait(1 - b)
            @pl.when(k >= 3)
            def _():
                out_wait(1 - b)
            add_scatter(1 - b)
            out_start(k - 1, 1 - b)

        @pl.when(chunk_of(k + 1) < c_end)
        def _():
            idx_start(k + 1, 1 - b)

        return carry

    def block_pair(m, carry):
        k = 2 * m
        block(k, 0, carry)
        block(k + 1, 1, carry)
        return carry

    lax.fori_loop(0, K_ITERS // 2 + 1, block_pair, 0)

    out_wait(0)
    out_wait(1)


_gather_sum = functools.partial(
    pl.kernel,
    mesh=plsc.VectorSubcoreMesh(core_axis_name="c", subcore_axis_name="s"),
    out_type=jax.ShapeDtypeStruct((OUT_ROWS, 128), jnp.float32),
    scratch_types=[
        pltpu.VMEM((2, 2 * CH_EDGES), jnp.int32),
        pltpu.VMEM((2, 2 * CH_EDGES, EMB), jnp.float32),
        pltpu.VMEM((2, 128, 129), jnp.float32),
        [pltpu.SemaphoreType.DMA] * 2,
        [pltpu.SemaphoreType.DMA] * 2,
        [pltpu.SemaphoreType.DMA] * 2,
    ],
    compiler_params=pltpu.CompilerParams(use_tc_tiling_on_sc=False,
                                         needs_layout_passes=False),
)(_body)


def kernel(edge_attr, W0, W1):
    if edge_attr.ndim == 1:
        edge_attr = edge_attr[:, None]
    x1d = (edge_attr.astype(jnp.int32)
           .reshape(N_EDGES // 128, 128, 2)
           .transpose(0, 2, 1)
           .reshape(-1))
    w = jnp.concatenate([W0, W1], axis=0)
    out2 = _gather_sum(x1d, w)
    return (out2.reshape(2, N_EDGES // 128, 8, 128)
            .transpose(1, 3, 0, 2)
            .reshape(N_EDGES, EMB))
